# Initial kernel scaffold; baseline (speedup 1.0000x reference)
import jax
import jax.numpy as jnp
from jax.experimental import pallas as pl


def kernel(selfies, tasks, values, property_mask, W_emb, W_heads, b_heads):
    raise NotImplementedError("write your pallas kernel here")


# speedup vs baseline: 5.5172x; 5.5172x over previous
import functools

import jax
import jax.numpy as jnp
from jax import lax
from jax.experimental import pallas as pl
from jax.experimental.pallas import tpu as pltpu
from jax.experimental.pallas import tpu_sc as plsc

_HP = 16
_NC = 2
_NS = 16
_NW = _NC * _NS
_L = 16


def _table_body(wemb_ref, whp_ref, bhp_ref, t_ref, c_ref):
    whp = whp_ref[...]
    t = lax.dot_general(
        wemb_ref[...], whp,
        dimension_numbers=(((1,), (0,)), ((), ())),
        precision=lax.Precision.HIGHEST,
        preferred_element_type=jnp.float32,
    )
    t_ref[...] = t + bhp_ref[...]
    c_ref[...] = jnp.broadcast_to(jnp.sum(whp, axis=0, keepdims=True), c_ref.shape)


def _make_sc_combine(n_tokens):
    chunk = n_tokens // _NW
    n_it = chunk // _L
    mesh = plsc.VectorSubcoreMesh(core_axis_name="c", subcore_axis_name="s")

    @functools.partial(
        pl.kernel,
        mesh=mesh,
        compiler_params=pltpu.CompilerParams(needs_layout_passes=False),
        out_type=jax.ShapeDtypeStruct((n_tokens,), jnp.float32),
        scratch_types=[
            pltpu.VMEM((chunk,), jnp.int32),
            pltpu.VMEM((chunk,), jnp.int32),
            pltpu.VMEM((chunk,), jnp.float32),
            pltpu.VMEM((chunk,), jnp.float32),
            pltpu.VMEM((512 * _HP,), jnp.float32),
            pltpu.VMEM((_HP,), jnp.float32),
            pltpu.VMEM((chunk,), jnp.float32),
        ],
    )
    def sc_combine(sel_hbm, tsk_hbm, val_hbm, msk_hbm, t_hbm, c_hbm, out_hbm,
                   sel_v, tsk_v, val_v, msk_v, t_v, c_v, out_v):
        wid = lax.axis_index("s") * _NC + lax.axis_index("c")
        base = wid * chunk
        pltpu.sync_copy(sel_hbm.at[pl.ds(base, chunk)], sel_v)
        pltpu.sync_copy(tsk_hbm.at[pl.ds(base, chunk)], tsk_v)
        pltpu.sync_copy(val_hbm.at[pl.ds(base, chunk)], val_v)
        pltpu.sync_copy(msk_hbm.at[pl.ds(base, chunk)], msk_v)
        pltpu.sync_copy(t_hbm, t_v)
        pltpu.sync_copy(c_hbm, c_v)

        def it(i, carry):
            ds = pl.ds(i * _L, _L)
            tsk = tsk_v[ds]
            flat = sel_v[ds] * _HP + tsk
            tval = plsc.load_gather(t_v, [flat])
            cval = plsc.load_gather(c_v, [tsk])
            out_v[ds] = tval + val_v[ds] * msk_v[ds] * cval
            return carry

        lax.fori_loop(0, n_it, it, 0)
        pltpu.sync_copy(out_v, out_hbm.at[pl.ds(base, chunk)])

    return sc_combine


def kernel(selfies, tasks, values, property_mask, W_emb, W_heads, b_heads):
    B, S = selfies.shape
    V, D = W_emb.shape
    H, _, O = W_heads.shape

    whp = jnp.pad(W_heads[:, :, 0].T, ((0, 0), (0, _HP - H)))
    bhp = jnp.pad(b_heads[:, 0][None, :], ((0, 0), (0, _HP - H)))

    t_tab, c_tab = pl.pallas_call(
        _table_body,
        out_shape=(
            jax.ShapeDtypeStruct((V, _HP), jnp.float32),
            jax.ShapeDtypeStruct((8, _HP), jnp.float32),
        ),
    )(W_emb, whp, bhp)

    n_tokens = B * S
    out_flat = _make_sc_combine(n_tokens)(
        selfies.reshape(-1).astype(jnp.int32),
        tasks.reshape(-1).astype(jnp.int32),
        values.reshape(-1),
        property_mask.reshape(-1),
        t_tab.reshape(-1),
        c_tab[0],
    )
    return out_flat.reshape(B, S, O)


# speedup vs baseline: 6.7263x; 1.2192x over previous
import functools

import jax
import jax.numpy as jnp
from jax import lax
from jax.experimental import pallas as pl
from jax.experimental.pallas import tpu as pltpu
from jax.experimental.pallas import tpu_sc as plsc

_NC = 2
_NS = 16
_NW = _NC * _NS
_L = 16


def _table_body(wemb_ref, wh_ref, bh_ref, t_ref):
    wh = wh_ref[...]
    dn = (((1,), (1,)), ((), ()))
    t = lax.dot_general(
        wemb_ref[...], wh, dimension_numbers=dn,
        precision=lax.Precision.HIGHEST, preferred_element_type=jnp.float32,
    )
    t_ref[0:512, :] = t + bh_ref[...]
    c = lax.dot_general(
        jnp.ones((1, wh.shape[1]), jnp.float32), wh, dimension_numbers=dn,
        precision=lax.Precision.HIGHEST, preferred_element_type=jnp.float32,
    )
    t_ref[512:520, :] = jnp.broadcast_to(c, (8, wh.shape[0]))


def _make_sc_combine(n_tokens, h):
    chunk = n_tokens // _NW
    n_it = chunk // _L
    t_rows = 520
    mesh = plsc.VectorSubcoreMesh(core_axis_name="c", subcore_axis_name="s")

    @functools.partial(
        pl.kernel,
        mesh=mesh,
        compiler_params=pltpu.CompilerParams(needs_layout_passes=False),
        out_type=jax.ShapeDtypeStruct((n_tokens,), jnp.float32),
        scratch_types=[
            pltpu.VMEM((chunk,), jnp.int32),
            pltpu.VMEM((chunk,), jnp.int32),
            pltpu.VMEM((chunk,), jnp.float32),
            pltpu.VMEM((chunk,), jnp.float32),
            pltpu.VMEM((t_rows * 8,), jnp.float32),
            pltpu.VMEM((chunk,), jnp.float32),
            pltpu.SemaphoreType.DMA,
        ],
    )
    def sc_combine(sel_hbm, tsk_hbm, val_hbm, msk_hbm, t_hbm, out_hbm,
                   sel_v, tsk_v, val_v, msk_v, t_v, out_v, sem):
        wid = lax.axis_index("s") * _NC + lax.axis_index("c")
        base = wid * chunk
        cps = [
            pltpu.async_copy(sel_hbm.at[pl.ds(base, chunk)], sel_v, sem),
            pltpu.async_copy(tsk_hbm.at[pl.ds(base, chunk)], tsk_v, sem),
            pltpu.async_copy(val_hbm.at[pl.ds(base, chunk)], val_v, sem),
            pltpu.async_copy(msk_hbm.at[pl.ds(base, chunk)], msk_v, sem),
            pltpu.async_copy(t_hbm, t_v, sem),
        ]
        for cp in cps:
            cp.wait()

        def it(i, carry):
            ds = pl.ds(i * _L, _L)
            tsk = tsk_v[ds]
            tval = plsc.load_gather(t_v, [sel_v[ds] * h + tsk])
            cval = plsc.load_gather(t_v, [tsk + 512 * h])
            out_v[ds] = tval + val_v[ds] * msk_v[ds] * cval
            return carry

        lax.fori_loop(0, n_it, it, 0)
        pltpu.sync_copy(out_v, out_hbm.at[pl.ds(base, chunk)])

    return sc_combine


def kernel(selfies, tasks, values, property_mask, W_emb, W_heads, b_heads):
    B, S = selfies.shape
    V, D = W_emb.shape
    H, _, O = W_heads.shape

    t_tab = pl.pallas_call(
        _table_body,
        out_shape=jax.ShapeDtypeStruct((520, H), jnp.float32),
    )(W_emb, W_heads.reshape(H, D), b_heads.reshape(1, H))

    n_tokens = B * S
    out_flat = _make_sc_combine(n_tokens, H)(
        selfies.reshape(-1).astype(jnp.int32),
        tasks.reshape(-1).astype(jnp.int32),
        values.reshape(-1),
        property_mask.reshape(-1),
        t_tab.reshape(-1),
    )
    return out_flat.reshape(B, S, O)


# speedup vs baseline: 6.8678x; 1.0210x over previous
import functools

import jax
import jax.numpy as jnp
from jax import lax
from jax.experimental import pallas as pl
from jax.experimental.pallas import tpu as pltpu
from jax.experimental.pallas import tpu_sc as plsc

_NC = 2
_NS = 16
_NW = _NC * _NS
_L = 16
_TCOLS = 640


def _table_body(wemb_ref, wh_ref, bh_ref, t_ref):
    wh = wh_ref[...]
    dn = (((1,), (1,)), ((), ()))
    t = lax.dot_general(
        wh, wemb_ref[...], dimension_numbers=dn,
        precision=lax.Precision.HIGHEST, preferred_element_type=jnp.float32,
    )
    t_ref[:, 0:512] = t + bh_ref[...]
    c = jnp.sum(wh, axis=1, keepdims=True)
    t_ref[:, 512:_TCOLS] = jnp.broadcast_to(c, (wh.shape[0], _TCOLS - 512))


def _make_sc_combine(B, S, h_rows):
    chunk = B * S // _NW
    per_row = S // chunk
    n_it = chunk // _L
    mesh = plsc.VectorSubcoreMesh(core_axis_name="c", subcore_axis_name="s")

    @functools.partial(
        pl.kernel,
        mesh=mesh,
        compiler_params=pltpu.CompilerParams(needs_layout_passes=False),
        out_type=jax.ShapeDtypeStruct((B, S), jnp.float32),
        scratch_types=[
            pltpu.VMEM((chunk,), jnp.int32),
            pltpu.VMEM((chunk,), jnp.int32),
            pltpu.VMEM((chunk,), jnp.float32),
            pltpu.VMEM((chunk,), jnp.float32),
            pltpu.VMEM((h_rows, _TCOLS), jnp.float32),
            pltpu.VMEM((chunk,), jnp.float32),
            pltpu.SemaphoreType.DMA,
        ],
    )
    def sc_combine(sel_hbm, tsk_hbm, val_hbm, msk_hbm, t_hbm, out_hbm,
                   sel_v, tsk_v, val_v, msk_v, t_v, out_v, sem):
        wid = lax.axis_index("s") * _NC + lax.axis_index("c")
        b = wid // per_row
        s0 = (wid % per_row) * chunk
        cps = [
            pltpu.async_copy(sel_hbm.at[b, pl.ds(s0, chunk)], sel_v, sem),
            pltpu.async_copy(tsk_hbm.at[b, pl.ds(s0, chunk)], tsk_v, sem),
            pltpu.async_copy(val_hbm.at[b, pl.ds(s0, chunk)], val_v, sem),
            pltpu.async_copy(msk_hbm.at[b, pl.ds(s0, chunk)], msk_v, sem),
            pltpu.async_copy(t_hbm, t_v, sem),
        ]
        for cp in cps:
            cp.wait()

        c_col = jnp.full((_L,), 512, jnp.int32)

        def it(i, carry):
            ds = pl.ds(i * _L, _L)
            tsk = tsk_v[ds]
            tval = plsc.load_gather(t_v, [tsk, sel_v[ds]])
            cval = plsc.load_gather(t_v, [tsk, c_col])
            out_v[ds] = tval + val_v[ds] * msk_v[ds] * cval
            return carry

        lax.fori_loop(0, n_it, it, 0)
        pltpu.sync_copy(out_v, out_hbm.at[b, pl.ds(s0, chunk)])

    return sc_combine


def kernel(selfies, tasks, values, property_mask, W_emb, W_heads, b_heads):
    B, S = selfies.shape
    V, D = W_emb.shape
    H, _, O = W_heads.shape

    t_tab = pl.pallas_call(
        _table_body,
        out_shape=jax.ShapeDtypeStruct((H, _TCOLS), jnp.float32),
    )(W_emb, W_heads.reshape(H, D), b_heads.reshape(H, 1))

    out2d = _make_sc_combine(B, S, H)(
        selfies.astype(jnp.int32),
        tasks.astype(jnp.int32),
        values,
        property_mask,
        t_tab,
    )
    return out2d.reshape(B, S, O)
